# Initial kernel scaffold; baseline (speedup 1.0000x reference)
import jax
import jax.numpy as jnp
from jax.experimental import pallas as pl


def kernel(bytes_input, W_byte, W_ng0, W_ng1, W_ng2, W_ng3, W_ng4, W_ng5):
    raise NotImplementedError("write your pallas kernel here")


# speedup vs baseline: 2.5387x; 2.5387x over previous
import functools

import jax
import jax.numpy as jnp
from jax import lax
from jax.experimental import pallas as pl
from jax.experimental.pallas import tpu as pltpu
from jax.experimental.pallas import tpu_sc as plsc

_B, _S, _H, _V = 4, 4096, 768, 100000
_N = _B * _S
_NC, _NS = 2, 16
_NW = _NC * _NS
_TPW = _N // _NW
_K = 16
_NCHUNK = _TPW // _K
_NVJ = _H // 16
_NT = 7


def _ngram_hash(bytes_input, n, num_embeddings):
    seq_length = bytes_input.shape[1]
    win = jnp.arange(seq_length - n + 1)[:, None] + jnp.arange(n)[None, :]
    ngrams = bytes_input[:, win]
    exponents = jnp.arange(n).astype(jnp.float32)
    weights = (256.0 ** exponents)[None, None, :]
    hash_values = (ngrams.astype(jnp.float32) * weights).sum(axis=-1).astype(jnp.int64)
    return jnp.mod(hash_values, num_embeddings)


def _sc_lookup_combine(idx, W_byte, W3, W4, W5, W6, W7, W8):
    mesh = plsc.VectorSubcoreMesh(core_axis_name="c", subcore_axis_name="s")

    @functools.partial(
        pl.kernel,
        mesh=mesh,
        out_type=jax.ShapeDtypeStruct((_N, _H), jnp.float32),
        scratch_types=(
            [pltpu.VMEM((_NCHUNK * _NT, _K), jnp.int32)]
            + [pltpu.VMEM((_K, _H), jnp.float32) for _ in range(_NT)]
            + [pltpu.VMEM((_K, _H), jnp.float32), pltpu.SemaphoreType.DMA]
        ),
    )
    def run(idx_hbm, wb, w3, w4, w5, w6, w7, w8, out_hbm,
            idxv, b0, b1, b2, b3, b4, b5, b6, outb, sem):
        tables = (wb, w3, w4, w5, w6, w7, w8)
        bufs = (b0, b1, b2, b3, b4, b5, b6)
        wid = lax.axis_index("s") * jnp.int32(_NC) + lax.axis_index("c")
        base = wid * jnp.int32(_TPW)
        pltpu.sync_copy(idx_hbm.at[wid], idxv)

        def chunk_body(ci, carry):
            cb = base + ci * jnp.int32(_K)
            cps = [
                pltpu.async_copy(
                    tables[t].at[idxv.at[ci * jnp.int32(_NT) + jnp.int32(t)]],
                    bufs[t], sem)
                for t in range(_NT)
            ]
            for cp in cps:
                cp.wait()

            def tok_body(i, _):
                pos = lax.rem(cb + i, jnp.int32(_S))
                posv = jnp.full((16,), pos, dtype=jnp.int32)
                scales = [
                    jnp.where(posv < (_S - n + 1),
                              jnp.float32(1.0 / n), jnp.float32(0.0))
                    for n in range(3, 9)
                ]

                def vec_body(j, __):
                    sl = pl.ds(j * jnp.int32(16), 16)
                    acc = bufs[0][i, sl]
                    for t in range(6):
                        acc = acc + scales[t] * bufs[t + 1][i, sl]
                    outb[i, sl] = acc
                    return __

                lax.fori_loop(jnp.int32(0), jnp.int32(_NVJ), vec_body, None)
                return _

            lax.fori_loop(jnp.int32(0), jnp.int32(_K), tok_body, None)
            pltpu.sync_copy(outb, out_hbm.at[pl.ds(cb, _K)])
            return carry

        lax.fori_loop(jnp.int32(0), jnp.int32(_NCHUNK), chunk_body, None)

    return run(idx, W_byte, W3, W4, W5, W6, W7, W8)


def kernel(bytes_input, W_byte, W_ng0, W_ng1, W_ng2, W_ng3, W_ng4, W_ng5):
    tables = [W_ng0, W_ng1, W_ng2, W_ng3, W_ng4, W_ng5]
    idx_list = [bytes_input.reshape(_N).astype(jnp.int32)]
    for n in range(3, 9):
        h = _ngram_hash(bytes_input, n, tables[n - 3].shape[0])
        h = jnp.pad(h, ((0, 0), (0, n - 1)))
        idx_list.append(h.reshape(_N).astype(jnp.int32))
    idx = jnp.stack(idx_list)
    idx = (idx.reshape(_NT, _NW, _NCHUNK, _K)
              .transpose(1, 2, 0, 3)
              .reshape(_NW, _NCHUNK * _NT, _K))
    out = _sc_lookup_combine(idx, W_byte, *tables)
    return out.reshape(_B, _S, _H)
